# baseline (speedup 1.0000x reference)
import functools

import jax
import jax.numpy as jnp
from jax.experimental import pallas as pl
from jax.experimental.pallas import tpu as pltpu

_BLOCK_ROWS = 1024


def _proj_body(y_ref, w_ref, b_ref, o_ref):
    z = jnp.dot(y_ref[...], w_ref[...], preferred_element_type=jnp.float32)
    o_ref[...] = jax.nn.sigmoid(z + b_ref[...])


def _project(y, lin_w, lin_b):
    n, f = y.shape
    npad = (-n) % _BLOCK_ROWS
    ypad = jnp.pad(y, ((0, npad), (0, 0)))
    grid = (ypad.shape[0] // _BLOCK_ROWS,)
    out = pl.pallas_call(
        _proj_body,
        grid=grid,
        in_specs=[
            pl.BlockSpec((_BLOCK_ROWS, f), lambda i: (i, 0)),
            pl.BlockSpec((f, f), lambda i: (0, 0)),
            pl.BlockSpec((1, f), lambda i: (0, 0)),
        ],
        out_specs=pl.BlockSpec((_BLOCK_ROWS, f), lambda i: (i, 0)),
        out_shape=jax.ShapeDtypeStruct((ypad.shape[0], f), jnp.float32),
    )(ypad, lin_w.T, lin_b[None, :])
    return out[:n]


def kernel(H_rows, H_cols, H_vals, feats, W_, lin_w, lin_b):
    N, _ = feats.shape
    E = W_.shape[0]
    w = W_[:, 0]
    d_v = jax.ops.segment_sum(H_vals * w[H_cols], H_rows, num_segments=N) ** -0.5
    d_e = jax.ops.segment_sum(H_vals, H_cols, num_segments=E) ** -1.0
    X = feats * d_v[:, None]
    M = jax.ops.segment_sum(H_vals[:, None] * X[H_rows], H_cols, num_segments=E)
    M = M * (w * d_e)[:, None]
    Y = jax.ops.segment_sum(H_vals[:, None] * M[H_cols], H_rows, num_segments=N)
    Y = Y * d_v[:, None]
    return _project(Y, lin_w, lin_b)
